# speedup vs baseline: 1.1207x; 1.1207x over previous
import jax
import jax.numpy as jnp
from jax import lax
from jax.experimental import pallas as pl

NUM_TOKENS = 32768
INPUT_DIM = 768
NUM_EXPERTS = 8
TOKEN_BLOCK = 2048
GRID = NUM_TOKENS // TOKEN_BLOCK


def _router_body(x_ref, w_ref, b_ref, idx_ref, wt_ref, cnt_ref):
    i = pl.program_id(0)
    x = x_ref[...]
    w = w_ref[...]
    b = b_ref[...]
    logits = lax.dot_general(
        x, w, dimension_numbers=(((1,), (1,)), ((), ())),
        preferred_element_type=jnp.float32,
    ) + b
    m = jnp.max(logits, axis=1, keepdims=True)
    unnorm = jnp.exp(logits - m)
    s = jnp.sum(unnorm, axis=1, keepdims=True)
    probs = unnorm / s
    pmax = jnp.max(probs, axis=1, keepdims=True)
    iota_e = lax.broadcasted_iota(jnp.int32, probs.shape, 1)
    idx = jnp.min(jnp.where(probs == pmax, iota_e, NUM_EXPERTS),
                  axis=1, keepdims=True)
    idx_ref[...] = idx
    wt_ref[...] = pmax
    onehot = (iota_e == idx).astype(jnp.float32)
    partial = jnp.sum(onehot, axis=0, keepdims=True)

    @pl.when(i == 0)
    def _():
        cnt_ref[...] = partial

    @pl.when(i > 0)
    def _():
        cnt_ref[...] = cnt_ref[...] + partial


def kernel(x, W, b):
    idx2d, wt2d, cnt2d = pl.pallas_call(
        _router_body,
        grid=(GRID,),
        in_specs=[
            pl.BlockSpec((TOKEN_BLOCK, INPUT_DIM), lambda i: (i, 0)),
            pl.BlockSpec((NUM_EXPERTS, INPUT_DIM), lambda i: (0, 0)),
            pl.BlockSpec((1, NUM_EXPERTS), lambda i: (0, 0)),
        ],
        out_specs=[
            pl.BlockSpec((TOKEN_BLOCK, 1), lambda i: (i, 0)),
            pl.BlockSpec((TOKEN_BLOCK, 1), lambda i: (i, 0)),
            pl.BlockSpec((1, NUM_EXPERTS), lambda i: (0, 0)),
        ],
        out_shape=[
            jax.ShapeDtypeStruct((NUM_TOKENS, 1), jnp.int32),
            jax.ShapeDtypeStruct((NUM_TOKENS, 1), jnp.float32),
            jax.ShapeDtypeStruct((1, NUM_EXPERTS), jnp.float32),
        ],
    )(x, W, b.reshape(1, NUM_EXPERTS))
    return idx2d[:, 0], wt2d[:, 0], cnt2d[0]
